# Initial kernel scaffold; baseline (speedup 1.0000x reference)
import jax
import jax.numpy as jnp
from jax.experimental import pallas as pl


def kernel(x, edge_index, edge_type, W1, root1, b1, W2, root2, b2):
    raise NotImplementedError("write your pallas kernel here")


# speedup vs baseline: 3.8779x; 3.8779x over previous
import functools

import jax
import jax.numpy as jnp
from jax import lax
from jax.experimental import pallas as pl
from jax.experimental.pallas import tpu as pltpu
from jax.experimental.pallas import tpu_sc as plsc

N = 10000
E = 320000
D = 128
R = 8

NSC = 2
NT = 16
CH = 128
CW = 8
NG = D // CW
GPC = NG // NSC

EPT_COL = E // NT
EPT_CNT = E // (NSC * NT)
NCH_COL = (EPT_COL + CH - 1) // CH
NCH_CNT = (EPT_CNT + CH - 1) // CH
WORDS_COL = NCH_COL * CH
WORDS_CNT = NCH_CNT * CH

SEGS = N * R
DUMP = SEGS
ACC_ROWS = 81920
ZROWS = 512
SEG_PT = SEGS // NT


def _make_sc(with_counts: bool):
    mesh = plsc.VectorSubcoreMesh(core_axis_name="c", subcore_axis_name="s")
    if with_counts:
        out_type = (jax.ShapeDtypeStruct((SEGS, NG, CW), jnp.float32),
                    jax.ShapeDtypeStruct((NSC, SEGS, CW), jnp.float32))
    else:
        out_type = jax.ShapeDtypeStruct((SEGS, NG, CW), jnp.float32)

    def body(xv, srcg, dstg, etyg, zrows_h, ones_h, *rest):
        if with_counts:
            sums_out, cnt_out = rest[0], rest[1]
            scratch = rest[2:]
        else:
            sums_out = rest[0]
            scratch = rest[1:]
        (dst1d, ety1d, src1d, seg2d, gidx1d, rows, ones, zeros, acc) = scratch

        c = lax.axis_index("c")
        t = lax.axis_index("s")

        pltpu.sync_copy(zrows_h, zeros)
        pltpu.sync_copy(ones_h, ones)

        def _stage(base_e, nreal, nch):
            pltpu.sync_copy(dstg.at[pl.ds(base_e, nreal)],
                            dst1d.at[pl.ds(0, nreal)])
            pltpu.sync_copy(etyg.at[pl.ds(base_e, nreal)],
                            ety1d.at[pl.ds(0, nreal)])
            pltpu.sync_copy(srcg.at[pl.ds(base_e, nreal)],
                            src1d.at[pl.ds(0, nreal)])

            def _segrow(r_, _):
                for k in range(CH // 16):
                    sl = pl.ds(r_ * CH + k * 16, 16)
                    seg2d[r_, pl.ds(k * 16, 16)] = (
                        ety1d[sl] * N + dst1d[sl])
                return 0
            lax.fori_loop(0, nch, _segrow, 0)
            lane0 = nreal - (nch - 1) * CH
            for k in range((nch * CH - nreal) // 16):
                seg2d[nch - 1, pl.ds(lane0 + k * 16, 16)] = (
                    jnp.full((16,), DUMP, jnp.int32))
                src1d[pl.ds(nreal + k * 16, 16)] = jnp.zeros((16,), jnp.int32)

        def _zero_acc():
            base = t * (ACC_ROWS // NT)
            for j in range(ACC_ROWS // NT // ZROWS):
                pltpu.sync_copy(zeros, acc.at[pl.ds(base + j * ZROWS, ZROWS)])

        if with_counts:
            _stage((c * NT + t) * EPT_CNT, EPT_CNT, NCH_CNT)
            _zero_acc()
            plsc.subcore_barrier()

            def _cch(ci, _):
                pltpu.sync_copy(ones, acc.at[seg2d.at[ci]], add=True)
                return 0
            lax.fori_loop(0, NCH_CNT, _cch, 0)
            plsc.subcore_barrier()
            pltpu.sync_copy(acc.at[pl.ds(t * SEG_PT, SEG_PT)],
                            cnt_out.at[c, pl.ds(t * SEG_PT, SEG_PT)])

        _stage(t * EPT_COL, EPT_COL, NCH_COL)
        for j in range(GPC):
            p = c * GPC + j
            plsc.subcore_barrier()
            _zero_acc()
            plsc.subcore_barrier()

            def _ech(ci, _):
                for k in range(CH // 16):
                    sl = pl.ds(ci * CH + k * 16, 16)
                    gidx1d[pl.ds(k * 16, 16)] = src1d[sl] * NG + p
                pltpu.sync_copy(xv.at[gidx1d], rows)
                pltpu.sync_copy(rows, acc.at[seg2d.at[ci]], add=True)
                return 0
            lax.fori_loop(0, NCH_COL, _ech, 0)
            plsc.subcore_barrier()
            pltpu.sync_copy(acc.at[pl.ds(t * SEG_PT, SEG_PT)],
                            sums_out.at[pl.ds(t * SEG_PT, SEG_PT), p])

    return pl.kernel(
        body,
        out_type=out_type,
        mesh=mesh,
        compiler_params=pltpu.CompilerParams(use_tc_tiling_on_sc=False),
        scratch_types=[
            pltpu.VMEM((WORDS_COL,), jnp.int32),
            pltpu.VMEM((WORDS_COL,), jnp.int32),
            pltpu.VMEM((WORDS_COL,), jnp.int32),
            pltpu.VMEM((NCH_COL, CH), jnp.int32),
            pltpu.VMEM((CH,), jnp.int32),
            pltpu.VMEM((CH, CW), jnp.float32),
            pltpu.VMEM((CH, CW), jnp.float32),
            pltpu.VMEM((ZROWS, CW), jnp.float32),
            pltpu.VMEM_SHARED((ACC_ROWS, CW), jnp.float32),
        ],
    )


_sc_agg_counts = _make_sc(True)
_sc_agg = _make_sc(False)

_BN = 1000


def _tc_body(relu, sums_ref, cnt0_ref, cnt1_ref, x_ref, w_ref, root_ref,
             b_ref, o_ref):
    acc = jnp.dot(x_ref[...], root_ref[...],
                  preferred_element_type=jnp.float32)
    for r in range(R):
        cnt = cnt0_ref[r, :, 0:1] + cnt1_ref[r, :, 0:1]
        inv = 1.0 / jnp.maximum(cnt, 1.0)
        acc = acc + jnp.dot(sums_ref[r] * inv, w_ref[r],
                            preferred_element_type=jnp.float32)
    acc = acc + b_ref[...]
    if relu:
        acc = jnp.maximum(acc, 0.0)
    o_ref[...] = acc


def _make_tc(relu: bool):
    return pl.pallas_call(
        functools.partial(_tc_body, relu),
        grid=(N // _BN,),
        in_specs=[
            pl.BlockSpec((R, _BN, D), lambda i: (0, i, 0)),
            pl.BlockSpec((R, _BN, CW), lambda i: (0, i, 0)),
            pl.BlockSpec((R, _BN, CW), lambda i: (0, i, 0)),
            pl.BlockSpec((_BN, D), lambda i: (i, 0)),
            pl.BlockSpec((R, D, D), lambda i: (0, 0, 0)),
            pl.BlockSpec((D, D), lambda i: (0, 0)),
            pl.BlockSpec((1, D), lambda i: (0, 0)),
        ],
        out_specs=pl.BlockSpec((_BN, D), lambda i: (i, 0)),
        out_shape=jax.ShapeDtypeStruct((N, D), jnp.float32),
    )


_tc_layer_relu = _make_tc(True)
_tc_layer = _make_tc(False)


def kernel(x, edge_index, edge_type, W1, root1, b1, W2, root2, b2):
    src = edge_index[0]
    dst = edge_index[1]

    zrows = jnp.zeros((ZROWS, CW), jnp.float32)
    onesb = jnp.ones((CH, CW), jnp.float32)
    sums1, cnt = _sc_agg_counts(x.reshape(N * NG, CW), src, dst, edge_type,
                                zrows, onesb)
    cnt0 = cnt[0].reshape(R, N, CW)
    cnt1 = cnt[1].reshape(R, N, CW)
    h = _tc_layer_relu(sums1.reshape(R, N, D), cnt0, cnt1, x, W1, root1,
                       b1.reshape(1, D))
    sums2 = _sc_agg(h.reshape(N * NG, CW), src, dst, edge_type, zrows, onesb)
    out = _tc_layer(sums2.reshape(R, N, D), cnt0, cnt1, h, W2, root2,
                    b2.reshape(1, D))
    return out


# speedup vs baseline: 5.6889x; 1.4670x over previous
import functools

import jax
import jax.numpy as jnp
from jax import lax
from jax.experimental import pallas as pl
from jax.experimental.pallas import tpu as pltpu
from jax.experimental.pallas import tpu_sc as plsc

N = 10000
E = 320000
D = 128
R = 8

NSC = 2
NT = 16
CH = 128
CW = 8
NG = D // CW
GPC = NG // NSC

CB = 512
CR = CB // CH
AHEAD = 4
NBUF = 2 * AHEAD

EPT_COL = E // NT
EPT_CNT = E // (NSC * NT)
NCH_COL = (EPT_COL + CB - 1) // CB
NCH_CNT = (EPT_CNT + CB - 1) // CB
NRND = NCH_COL // AHEAD

SEGS = N * R
DUMP = SEGS
ACC_ROWS = 81920
ZROWS = 256
SEG_PT = SEGS // NT


def _make_sc(with_counts: bool):
    mesh = plsc.VectorSubcoreMesh(core_axis_name="c", subcore_axis_name="s")
    if with_counts:
        out_type = (jax.ShapeDtypeStruct((SEGS, NG, CW), jnp.float32),
                    jax.ShapeDtypeStruct((NSC, SEGS, CW), jnp.float32))
    else:
        out_type = jax.ShapeDtypeStruct((SEGS, NG, CW), jnp.float32)

    def body(xv2, srcg, dstg, etyg, zrows_h, ones_h, *rest):
        if with_counts:
            sums_out, cnt_out = rest[0], rest[1]
            scratch = rest[2:]
        else:
            sums_out = rest[0]
            scratch = rest[1:]
        (slab_s, slab_d, slab_e, gbase, seg3, ones, zeros) = scratch[:7]
        rows = scratch[7:7 + NBUF]
        gsem = scratch[7 + NBUF:7 + 2 * NBUF]
        ssem = scratch[7 + 2 * NBUF:7 + 3 * NBUF]

        c = lax.axis_index("c")
        t = lax.axis_index("s")

        pltpu.sync_copy(zrows_h, zeros)
        pltpu.sync_copy(ones_h, ones)

        def _stage_rows(base_e, q, nwords, goff):
            want_src = goff is not None
            if want_src:
                pltpu.sync_copy(srcg.at[pl.ds(base_e, nwords)],
                                slab_s.at[pl.ds(0, nwords)])
            pltpu.sync_copy(dstg.at[pl.ds(base_e, nwords)],
                            slab_d.at[pl.ds(0, nwords)])
            pltpu.sync_copy(etyg.at[pl.ds(base_e, nwords)],
                            slab_e.at[pl.ds(0, nwords)])
            for jj in range(nwords // 16):
                sl = pl.ds(jj * 16, 16)
                if want_src:
                    gbase[q, 0, sl] = slab_s[sl] * NG + goff
                seg3[q, 0, sl] = slab_e[sl] * N + slab_d[sl]
            for jj in range(nwords // 16, CB // 16):
                sl = pl.ds(jj * 16, 16)
                if want_src:
                    gbase[q, 0, sl] = jnp.zeros((16,), jnp.int32)
                seg3[q, 0, sl] = jnp.full((16,), DUMP, jnp.int32)

        def _stage(base_e, nreal, goff):
            nfull = nreal // CB
            tail = nreal - nfull * CB

            def _row(q, _):
                _stage_rows(base_e + q * CB, q, CB, goff)
                return 0
            lax.fori_loop(0, nfull, _row, 0)
            if tail:
                _stage_rows(base_e + nfull * CB, nfull, tail, goff)

        def _zero_acc():
            base = t * (ACC_ROWS // NT)
            for j in range(ACC_ROWS // NT // ZROWS):
                pltpu.sync_copy(zeros, acc_ref.at[pl.ds(base + j * ZROWS, ZROWS)])

        acc_ref = scratch[7 + 3 * NBUF]

        if with_counts:
            _stage((c * NT + t) * EPT_CNT, EPT_CNT, None)
            _zero_acc()
            plsc.subcore_barrier()

            def _cfire(ci, _):
                pltpu.async_copy(ones, acc_ref.at[seg3.at[ci, 0]], gsem[0],
                                 add=True)
                return 0
            lax.fori_loop(0, NCH_CNT, _cfire, 0)

            def _cdrain(ci, _):
                pltpu.make_async_copy(ones, acc_ref.at[seg3.at[0, 0]],
                                      gsem[0]).wait()
                return 0
            lax.fori_loop(0, NCH_CNT, _cdrain, 0)
            plsc.subcore_barrier()
            pltpu.sync_copy(acc_ref.at[pl.ds(t * SEG_PT, SEG_PT)],
                            cnt_out.at[c, pl.ds(t * SEG_PT, SEG_PT)])

        _stage(t * EPT_COL, EPT_COL, c * GPC)
        for j in range(GPC):
            p = c * GPC + j
            plsc.subcore_barrier()
            _zero_acc()
            if j > 0:
                def _gx(q, _):
                    for k in range(CB // 16):
                        sl = pl.ds(k * 16, 16)
                        gbase[q, 0, sl] = gbase[q, 0, sl] + 1
                    return 0
                lax.fori_loop(0, NCH_COL, _gx, 0)
            plsc.subcore_barrier()

            def _gather(ci, B):
                pltpu.async_copy(xv2.at[gbase.at[ci, 0]], rows[B], gsem[B])

            def _gwait(ci, B):
                pltpu.make_async_copy(xv2.at[gbase.at[ci, 0]], rows[B],
                                      gsem[B]).wait()

            def _scat(ci, B):
                pltpu.async_copy(rows[B], acc_ref.at[seg3.at[ci, 0]], ssem[B],
                                 add=True)

            def _swait(B):
                pltpu.make_async_copy(rows[B], acc_ref.at[seg3.at[0, 0]],
                                      ssem[B]).wait()

            for b in range(AHEAD):
                _gather(b, b)

            def _round(rd, _):
                for b in range(AHEAD):
                    ci = rd * AHEAD + b

                    def _slot(B, B2, ci=ci, rd=rd):
                        _gwait(ci, B)
                        _scat(ci, B)

                        @pl.when(rd < NRND - 1)
                        def _issue_next():
                            @pl.when(rd > 0)
                            def _w():
                                _swait(B2)
                            _gather(ci + AHEAD, B2)

                    @pl.when(rd % 2 == 0)
                    def _even():
                        _slot(b, b + AHEAD)

                    @pl.when(rd % 2 == 1)
                    def _odd():
                        _slot(b + AHEAD, b)
                return 0
            lax.fori_loop(0, NRND, _round, 0)
            for b in range(NBUF):
                _swait(b)
            plsc.subcore_barrier()
            pltpu.sync_copy(acc_ref.at[pl.ds(t * SEG_PT, SEG_PT)],
                            sums_out.at[pl.ds(t * SEG_PT, SEG_PT), p])

    return pl.kernel(
        body,
        out_type=out_type,
        mesh=mesh,
        compiler_params=pltpu.CompilerParams(use_tc_tiling_on_sc=False),
        scratch_types=(
            [
                pltpu.VMEM((CB,), jnp.int32),
                pltpu.VMEM((CB,), jnp.int32),
                pltpu.VMEM((CB,), jnp.int32),
                pltpu.VMEM((NCH_COL, 1, CB), jnp.int32),
                pltpu.VMEM((NCH_COL, 1, CB), jnp.int32),
                pltpu.VMEM((CB, CW), jnp.float32),
                pltpu.VMEM((ZROWS, CW), jnp.float32),
            ]
            + [pltpu.VMEM((CB, CW), jnp.float32) for _ in range(NBUF)]
            + [pltpu.SemaphoreType.DMA for _ in range(2 * NBUF)]
            + [pltpu.VMEM_SHARED((ACC_ROWS, CW), jnp.float32)]
        ),
    )


_sc_agg_counts = _make_sc(True)
_sc_agg = _make_sc(False)

_BN = 1000


def _tc_body(relu, sums_ref, cnt0_ref, cnt1_ref, x_ref, w_ref, root_ref,
             b_ref, o_ref):
    acc = jnp.dot(x_ref[...], root_ref[...],
                  preferred_element_type=jnp.float32)
    for r in range(R):
        cnt = cnt0_ref[r, :, 0:1] + cnt1_ref[r, :, 0:1]
        inv = 1.0 / jnp.maximum(cnt, 1.0)
        acc = acc + jnp.dot(sums_ref[r] * inv, w_ref[r],
                            preferred_element_type=jnp.float32)
    acc = acc + b_ref[...]
    if relu:
        acc = jnp.maximum(acc, 0.0)
    o_ref[...] = acc


def _make_tc(relu: bool):
    return pl.pallas_call(
        functools.partial(_tc_body, relu),
        grid=(N // _BN,),
        in_specs=[
            pl.BlockSpec((R, _BN, D), lambda i: (0, i, 0)),
            pl.BlockSpec((R, _BN, CW), lambda i: (0, i, 0)),
            pl.BlockSpec((R, _BN, CW), lambda i: (0, i, 0)),
            pl.BlockSpec((_BN, D), lambda i: (i, 0)),
            pl.BlockSpec((R, D, D), lambda i: (0, 0, 0)),
            pl.BlockSpec((D, D), lambda i: (0, 0)),
            pl.BlockSpec((1, D), lambda i: (0, 0)),
        ],
        out_specs=pl.BlockSpec((_BN, D), lambda i: (i, 0)),
        out_shape=jax.ShapeDtypeStruct((N, D), jnp.float32),
    )


_tc_layer_relu = _make_tc(True)
_tc_layer = _make_tc(False)


def kernel(x, edge_index, edge_type, W1, root1, b1, W2, root2, b2):
    src = edge_index[0]
    dst = edge_index[1]

    zrows = jnp.zeros((ZROWS, CW), jnp.float32)
    onesb = jnp.ones((CB, CW), jnp.float32)
    sums1, cnt = _sc_agg_counts(x.reshape(N * NG, CW), src, dst, edge_type,
                                zrows, onesb)
    cnt0 = cnt[0].reshape(R, N, CW)
    cnt1 = cnt[1].reshape(R, N, CW)
    h = _tc_layer_relu(sums1.reshape(R, N, D), cnt0, cnt1, x, W1, root1,
                       b1.reshape(1, D))
    sums2 = _sc_agg(h.reshape(N * NG, CW), src, dst, edge_type, zrows, onesb)
    out = _tc_layer(sums2.reshape(R, N, D), cnt0, cnt1, h, W2, root2,
                    b2.reshape(1, D))
    return out


# speedup vs baseline: 7.9459x; 1.3967x over previous
import functools

import jax
import jax.numpy as jnp
from jax import lax
from jax.experimental import pallas as pl
from jax.experimental.pallas import tpu as pltpu
from jax.experimental.pallas import tpu_sc as plsc

N = 10000
E = 320000
D = 128
R = 8

NSC = 2
NT = 16
CH = 128
CW = 16
NG = D // CW
GPC = NG // NSC

CB = 512
CR = CB // CH
AHEAD = 4
NBUF = 2 * AHEAD

EPT_COL = E // NT
EPT_CNT = E // (NSC * NT)
NCH_COL = (EPT_COL + CB - 1) // CB
NCH_CNT = (EPT_CNT + CB - 1) // CB
NRND = NCH_COL // AHEAD

SEGS = N * R
DUMP = SEGS
ACC_ROWS = 81920
ZROWS = 256
SEG_PT = SEGS // NT


def _make_sc(with_counts: bool):
    mesh = plsc.VectorSubcoreMesh(core_axis_name="c", subcore_axis_name="s")
    if with_counts:
        out_type = (jax.ShapeDtypeStruct((SEGS, NG, CW), jnp.bfloat16),
                    jax.ShapeDtypeStruct((NSC, SEGS, CW), jnp.bfloat16))
    else:
        out_type = jax.ShapeDtypeStruct((SEGS, NG, CW), jnp.bfloat16)

    def body(xv2, srcg, dstg, etyg, zrows_h, ones_h, *rest):
        if with_counts:
            sums_out, cnt_out = rest[0], rest[1]
            scratch = rest[2:]
        else:
            sums_out = rest[0]
            scratch = rest[1:]
        (slab_s, slab_d, slab_e, gbase, seg3, ones, zeros) = scratch[:7]
        rows = scratch[7:7 + NBUF]
        gsem = scratch[7 + NBUF:7 + 2 * NBUF]
        ssem = scratch[7 + 2 * NBUF:7 + 3 * NBUF]

        c = lax.axis_index("c")
        t = lax.axis_index("s")

        pltpu.sync_copy(zrows_h, zeros)
        pltpu.sync_copy(ones_h, ones)

        def _stage_rows(base_e, q, nwords, goff):
            want_src = goff is not None
            if want_src:
                pltpu.sync_copy(srcg.at[pl.ds(base_e, nwords)],
                                slab_s.at[pl.ds(0, nwords)])
            pltpu.sync_copy(dstg.at[pl.ds(base_e, nwords)],
                            slab_d.at[pl.ds(0, nwords)])
            pltpu.sync_copy(etyg.at[pl.ds(base_e, nwords)],
                            slab_e.at[pl.ds(0, nwords)])
            for jj in range(nwords // 16):
                sl = pl.ds(jj * 16, 16)
                if want_src:
                    gbase[q, 0, sl] = slab_s[sl] * NG + goff
                seg3[q, 0, sl] = slab_e[sl] * N + slab_d[sl]
            for jj in range(nwords // 16, CB // 16):
                sl = pl.ds(jj * 16, 16)
                if want_src:
                    gbase[q, 0, sl] = jnp.zeros((16,), jnp.int32)
                seg3[q, 0, sl] = jnp.full((16,), DUMP, jnp.int32)

        def _stage(base_e, nreal, goff):
            nfull = nreal // CB
            tail = nreal - nfull * CB

            def _row(q, _):
                _stage_rows(base_e + q * CB, q, CB, goff)
                return 0
            lax.fori_loop(0, nfull, _row, 0)
            if tail:
                _stage_rows(base_e + nfull * CB, nfull, tail, goff)

        def _zero_acc():
            base = t * (ACC_ROWS // NT)
            for j in range(ACC_ROWS // NT // ZROWS):
                pltpu.sync_copy(zeros, acc_ref.at[pl.ds(base + j * ZROWS, ZROWS)])

        acc_ref = scratch[7 + 3 * NBUF]

        if with_counts:
            _stage((c * NT + t) * EPT_CNT, EPT_CNT, None)
            _zero_acc()
            plsc.subcore_barrier()

            def _cfire(ci, _):
                pltpu.async_copy(ones, acc_ref.at[seg3.at[ci, 0]], gsem[0],
                                 add=True)
                return 0
            lax.fori_loop(0, NCH_CNT, _cfire, 0)

            def _cdrain(ci, _):
                pltpu.make_async_copy(ones, acc_ref.at[seg3.at[0, 0]],
                                      gsem[0]).wait()
                return 0
            lax.fori_loop(0, NCH_CNT, _cdrain, 0)
            plsc.subcore_barrier()
            pltpu.sync_copy(acc_ref.at[pl.ds(t * SEG_PT, SEG_PT)],
                            cnt_out.at[c, pl.ds(t * SEG_PT, SEG_PT)])

        _stage(t * EPT_COL, EPT_COL, c * GPC)
        for j in range(GPC):
            p = c * GPC + j
            plsc.subcore_barrier()
            _zero_acc()
            if j > 0:
                def _gx(q, _):
                    for k in range(CB // 16):
                        sl = pl.ds(k * 16, 16)
                        gbase[q, 0, sl] = gbase[q, 0, sl] + 1
                    return 0
                lax.fori_loop(0, NCH_COL, _gx, 0)
            plsc.subcore_barrier()

            def _gather(ci, B):
                pltpu.async_copy(xv2.at[gbase.at[ci, 0]], rows[B], gsem[B])

            def _gwait(ci, B):
                pltpu.make_async_copy(xv2.at[gbase.at[ci, 0]], rows[B],
                                      gsem[B]).wait()

            def _scat(ci, B):
                pltpu.async_copy(rows[B], acc_ref.at[seg3.at[ci, 0]], ssem[B],
                                 add=True)

            def _swait(B):
                pltpu.make_async_copy(rows[B], acc_ref.at[seg3.at[0, 0]],
                                      ssem[B]).wait()

            for b in range(AHEAD):
                _gather(b, b)

            def _round(rd, _):
                for b in range(AHEAD):
                    ci = rd * AHEAD + b

                    def _slot(B, B2, ci=ci, rd=rd):
                        _gwait(ci, B)
                        _scat(ci, B)

                        @pl.when(rd < NRND - 1)
                        def _issue_next():
                            @pl.when(rd > 0)
                            def _w():
                                _swait(B2)
                            _gather(ci + AHEAD, B2)

                    @pl.when(rd % 2 == 0)
                    def _even():
                        _slot(b, b + AHEAD)

                    @pl.when(rd % 2 == 1)
                    def _odd():
                        _slot(b + AHEAD, b)
                return 0
            lax.fori_loop(0, NRND, _round, 0)
            for b in range(NBUF):
                _swait(b)
            plsc.subcore_barrier()
            pltpu.sync_copy(acc_ref.at[pl.ds(t * SEG_PT, SEG_PT)],
                            sums_out.at[pl.ds(t * SEG_PT, SEG_PT), p])

    return pl.kernel(
        body,
        out_type=out_type,
        mesh=mesh,
        compiler_params=pltpu.CompilerParams(use_tc_tiling_on_sc=False),
        scratch_types=(
            [
                pltpu.VMEM((CB,), jnp.int32),
                pltpu.VMEM((CB,), jnp.int32),
                pltpu.VMEM((CB,), jnp.int32),
                pltpu.VMEM((NCH_COL, 1, CB), jnp.int32),
                pltpu.VMEM((NCH_COL, 1, CB), jnp.int32),
                pltpu.VMEM((CB, CW), jnp.bfloat16),
                pltpu.VMEM((ZROWS, CW), jnp.bfloat16),
            ]
            + [pltpu.VMEM((CB, CW), jnp.bfloat16) for _ in range(NBUF)]
            + [pltpu.SemaphoreType.DMA for _ in range(2 * NBUF)]
            + [pltpu.VMEM_SHARED((ACC_ROWS, CW), jnp.bfloat16)]
        ),
    )


_sc_agg_counts = _make_sc(True)
_sc_agg = _make_sc(False)

_BN = 2000


def _tc_body(relu, sums_ref, cnt0_ref, cnt1_ref, x_ref, w_ref, root_ref,
             b_ref, o_ref):
    acc = jnp.dot(x_ref[...], root_ref[...],
                  preferred_element_type=jnp.float32)
    for r in range(R):
        cnt = (cnt0_ref[r, :, 0:1].astype(jnp.float32)
               + cnt1_ref[r, :, 0:1].astype(jnp.float32))
        inv = 1.0 / jnp.maximum(cnt, 1.0)
        acc = acc + jnp.dot(sums_ref[r].astype(jnp.float32) * inv, w_ref[r],
                            preferred_element_type=jnp.float32)
    acc = acc + b_ref[...]
    if relu:
        acc = jnp.maximum(acc, 0.0)
    o_ref[...] = acc


def _make_tc(relu: bool):
    return pl.pallas_call(
        functools.partial(_tc_body, relu),
        grid=(N // _BN,),
        in_specs=[
            pl.BlockSpec((R, _BN, D), lambda i: (0, i, 0)),
            pl.BlockSpec((R, _BN, CW), lambda i: (0, i, 0)),
            pl.BlockSpec((R, _BN, CW), lambda i: (0, i, 0)),
            pl.BlockSpec((_BN, D), lambda i: (i, 0)),
            pl.BlockSpec((R, D, D), lambda i: (0, 0, 0)),
            pl.BlockSpec((D, D), lambda i: (0, 0)),
            pl.BlockSpec((1, D), lambda i: (0, 0)),
        ],
        out_specs=pl.BlockSpec((_BN, D), lambda i: (i, 0)),
        out_shape=jax.ShapeDtypeStruct((N, D), jnp.float32),
    )


_tc_layer_relu = _make_tc(True)
_tc_layer = _make_tc(False)


def kernel(x, edge_index, edge_type, W1, root1, b1, W2, root2, b2):
    src = edge_index[0]
    dst = edge_index[1]

    zrows = jnp.zeros((ZROWS, CW), jnp.bfloat16)
    onesb = jnp.ones((CB, CW), jnp.bfloat16)
    xb = x.astype(jnp.bfloat16)
    sums1, cnt = _sc_agg_counts(xb.reshape(N * NG, CW), src, dst, edge_type,
                                zrows, onesb)
    cnt0 = cnt[0].reshape(R, N, CW)
    cnt1 = cnt[1].reshape(R, N, CW)
    h = _tc_layer_relu(sums1.reshape(R, N, D), cnt0, cnt1, x, W1, root1,
                       b1.reshape(1, D))
    hb = h.astype(jnp.bfloat16)
    sums2 = _sc_agg(hb.reshape(N * NG, CW), src, dst, edge_type, zrows, onesb)
    out = _tc_layer(sums2.reshape(R, N, D), cnt0, cnt1, h, W2, root2,
                    b2.reshape(1, D))
    return out


# speedup vs baseline: 8.5679x; 1.0783x over previous
import functools

import jax
import jax.numpy as jnp
from jax import lax
from jax.experimental import pallas as pl
from jax.experimental.pallas import tpu as pltpu
from jax.experimental.pallas import tpu_sc as plsc

N = 10000
E = 320000
D = 128
R = 8

NSC = 2
NT = 16
CH = 128
CW = 16
NG = D // CW
GPC = NG // NSC

CB = 512
BLK = 5120
CR = CB // CH
AHEAD = 4
NBUF = 2 * AHEAD

EPT_COL = E // NT
EPT_CNT = E // (NSC * NT)
NCH_COL = (EPT_COL + CB - 1) // CB
NCH_CNT = (EPT_CNT + CB - 1) // CB
NRND = NCH_COL // AHEAD

SEGS = N * R
DUMP = SEGS
ACC_ROWS = 81920
ZROWS = 256
SEG_PT = SEGS // NT


def _make_sc(with_counts: bool):
    mesh = plsc.VectorSubcoreMesh(core_axis_name="c", subcore_axis_name="s")
    if with_counts:
        out_type = (jax.ShapeDtypeStruct((SEGS, NG, CW), jnp.bfloat16),
                    jax.ShapeDtypeStruct((NSC, SEGS, CW), jnp.bfloat16))
    else:
        out_type = jax.ShapeDtypeStruct((SEGS, NG, CW), jnp.bfloat16)

    def body(xv2, srcg, dstg, etyg, zacc_h, ones_h, *rest):
        if with_counts:
            sums_out, cnt_out = rest[0], rest[1]
            scratch = rest[2:]
        else:
            sums_out = rest[0]
            scratch = rest[1:]
        (stg, gbase, seg3) = scratch[:3]
        rows = scratch[3:3 + NBUF]
        gsem = scratch[3 + NBUF:3 + 2 * NBUF]
        ssem = scratch[3 + 2 * NBUF:3 + 3 * NBUF]

        c = lax.axis_index("c")
        t = lax.axis_index("s")


        def _stage(base_e, nreal, goff):
            nfull = nreal // CB
            ntail = (nreal - nfull * CB) // 16
            CPR = CB // 16
            blocks = []
            done = 0
            while done < nreal:
                take = min(BLK, nreal - done)
                blocks.append((done, take))
                done += take

            def _pass(hbm, write):
                for boff, take in blocks:
                    pltpu.sync_copy(hbm.at[pl.ds(base_e + boff, take)],
                                    stg.at[pl.ds(0, take)])
                    row0 = boff // CB
                    full = take // CB

                    def _r(q, _):
                        for k in range(CPR):
                            write(row0 + q, pl.ds(k * 16, 16),
                                  stg[pl.ds(q * CB + k * 16, 16)])
                        return 0
                    lax.fori_loop(0, full, _r, 0)
                    for k in range((take - full * CB) // 16):
                        write(row0 + full, pl.ds(k * 16, 16),
                              stg[pl.ds(full * CB + k * 16, 16)])

            def _wety(q, sl, v):
                seg3[q, 0, sl] = v * N

            def _wdst(q, sl, v):
                seg3[q, 0, sl] = seg3[q, 0, sl] + v

            _pass(etyg, _wety)
            _pass(dstg, _wdst)
            for k in range(ntail, CB // 16):
                seg3[nfull, 0, pl.ds(k * 16, 16)] = (
                    jnp.full((16,), DUMP, jnp.int32))
            if goff is not None:
                def _wsrc(q, sl, v):
                    gbase[q, 0, sl] = v * NG + goff
                _pass(srcg, _wsrc)
                for k in range(ntail, CB // 16):
                    gbase[nfull, 0, pl.ds(k * 16, 16)] = (
                        jnp.zeros((16,), jnp.int32))

        def _zero_start():
            pltpu.async_copy(
                zacc_h, acc_ref.at[pl.ds(t * (ACC_ROWS // NT), ACC_ROWS // NT)],
                gsem[0])

        def _zero_wait():
            pltpu.make_async_copy(
                zacc_h, acc_ref.at[pl.ds(t * (ACC_ROWS // NT), ACC_ROWS // NT)],
                gsem[0]).wait()

        acc_ref = scratch[3 + 3 * NBUF]

        if with_counts:
            _zero_start()
            pltpu.sync_copy(ones_h, rows[0])
            _stage((c * NT + t) * EPT_CNT, EPT_CNT, None)
            _zero_wait()
            plsc.subcore_barrier()

            def _cfire(ci, _):
                pltpu.async_copy(rows[0], acc_ref.at[seg3.at[ci, 0]], gsem[0],
                                 add=True)
                return 0
            lax.fori_loop(0, NCH_CNT, _cfire, 0)

            def _cdrain(ci, _):
                pltpu.make_async_copy(rows[0], acc_ref.at[seg3.at[0, 0]],
                                      gsem[0]).wait()
                return 0
            lax.fori_loop(0, NCH_CNT, _cdrain, 0)
            plsc.subcore_barrier()
            pltpu.sync_copy(acc_ref.at[pl.ds(t * SEG_PT, SEG_PT)],
                            cnt_out.at[c, pl.ds(t * SEG_PT, SEG_PT)])
            plsc.subcore_barrier()

        _stage(t * EPT_COL, EPT_COL, c * GPC)
        for j in range(GPC):
            p = c * GPC + j
            plsc.subcore_barrier()
            _zero_start()
            if j > 0:
                def _gx(q, _):
                    for k in range(CB // 16):
                        sl = pl.ds(k * 16, 16)
                        gbase[q, 0, sl] = gbase[q, 0, sl] + 1
                    return 0
                lax.fori_loop(0, NCH_COL, _gx, 0)
            _zero_wait()
            plsc.subcore_barrier()

            def _gather(ci, B):
                pltpu.async_copy(xv2.at[gbase.at[ci, 0]], rows[B], gsem[B])

            def _gwait(ci, B):
                pltpu.make_async_copy(xv2.at[gbase.at[ci, 0]], rows[B],
                                      gsem[B]).wait()

            def _scat(ci, B):
                pltpu.async_copy(rows[B], acc_ref.at[seg3.at[ci, 0]], ssem[B],
                                 add=True)

            def _swait(B):
                pltpu.make_async_copy(rows[B], acc_ref.at[seg3.at[0, 0]],
                                      ssem[B]).wait()

            for b in range(AHEAD):
                _gather(b, b)

            def _round(rd, _):
                for b in range(AHEAD):
                    ci = rd * AHEAD + b

                    def _slot(B, B2, ci=ci, rd=rd):
                        _gwait(ci, B)
                        _scat(ci, B)

                        @pl.when(rd < NRND - 1)
                        def _issue_next():
                            @pl.when(rd > 0)
                            def _w():
                                _swait(B2)
                            _gather(ci + AHEAD, B2)

                    @pl.when(rd % 2 == 0)
                    def _even():
                        _slot(b, b + AHEAD)

                    @pl.when(rd % 2 == 1)
                    def _odd():
                        _slot(b + AHEAD, b)
                return 0
            lax.fori_loop(0, NRND, _round, 0)
            for b in range(NBUF):
                _swait(b)
            plsc.subcore_barrier()
            pltpu.sync_copy(acc_ref.at[pl.ds(t * SEG_PT, SEG_PT)],
                            sums_out.at[pl.ds(t * SEG_PT, SEG_PT), p])

    return pl.kernel(
        body,
        out_type=out_type,
        mesh=mesh,
        compiler_params=pltpu.CompilerParams(use_tc_tiling_on_sc=False),
        scratch_types=(
            [
                pltpu.VMEM((BLK,), jnp.int32),
                pltpu.VMEM((NCH_COL, 1, CB), jnp.int32),
                pltpu.VMEM((NCH_COL, 1, CB), jnp.int32),
            ]
            + [pltpu.VMEM((CB, CW), jnp.bfloat16) for _ in range(NBUF)]
            + [pltpu.SemaphoreType.DMA for _ in range(2 * NBUF)]
            + [pltpu.VMEM_SHARED((ACC_ROWS, CW), jnp.bfloat16)]
        ),
    )


_sc_agg_counts = _make_sc(True)
_sc_agg = _make_sc(False)

_BN = 2000


def _tc_body(relu, sums_ref, cnt0_ref, cnt1_ref, x_ref, w_ref, root_ref,
             b_ref, o_ref):
    acc = jnp.dot(x_ref[...], root_ref[...],
                  preferred_element_type=jnp.float32)
    for r in range(R):
        cnt = (cnt0_ref[r, :, 0:1].astype(jnp.float32)
               + cnt1_ref[r, :, 0:1].astype(jnp.float32))
        inv = 1.0 / jnp.maximum(cnt, 1.0)
        acc = acc + jnp.dot(sums_ref[r].astype(jnp.float32) * inv, w_ref[r],
                            preferred_element_type=jnp.float32)
    acc = acc + b_ref[...]
    if relu:
        acc = jnp.maximum(acc, 0.0)
    o_ref[...] = acc


def _make_tc(relu: bool):
    return pl.pallas_call(
        functools.partial(_tc_body, relu),
        grid=(N // _BN,),
        in_specs=[
            pl.BlockSpec((R, _BN, D), lambda i: (0, i, 0)),
            pl.BlockSpec((R, _BN, CW), lambda i: (0, i, 0)),
            pl.BlockSpec((R, _BN, CW), lambda i: (0, i, 0)),
            pl.BlockSpec((_BN, D), lambda i: (i, 0)),
            pl.BlockSpec((R, D, D), lambda i: (0, 0, 0)),
            pl.BlockSpec((D, D), lambda i: (0, 0)),
            pl.BlockSpec((1, D), lambda i: (0, 0)),
        ],
        out_specs=pl.BlockSpec((_BN, D), lambda i: (i, 0)),
        out_shape=jax.ShapeDtypeStruct((N, D), jnp.float32),
    )


_tc_layer_relu = _make_tc(True)
_tc_layer = _make_tc(False)


def kernel(x, edge_index, edge_type, W1, root1, b1, W2, root2, b2):
    src = edge_index[0]
    dst = edge_index[1]

    zacc = jnp.zeros((ACC_ROWS // NT, CW), jnp.bfloat16)
    onesb = jnp.ones((CB, CW), jnp.bfloat16)
    xb = x.astype(jnp.bfloat16)
    sums1, cnt = _sc_agg_counts(xb.reshape(N * NG, CW), src, dst, edge_type,
                                zacc, onesb)
    cnt0 = cnt[0].reshape(R, N, CW)
    cnt1 = cnt[1].reshape(R, N, CW)
    h = _tc_layer_relu(sums1.reshape(R, N, D), cnt0, cnt1, x, W1, root1,
                       b1.reshape(1, D))
    hb = h.astype(jnp.bfloat16)
    sums2 = _sc_agg(hb.reshape(N * NG, CW), src, dst, edge_type, zacc, onesb)
    out = _tc_layer(sums2.reshape(R, N, D), cnt0, cnt1, h, W2, root2,
                    b2.reshape(1, D))
    return out


# speedup vs baseline: 8.6311x; 1.0074x over previous
import functools

import jax
import jax.numpy as jnp
from jax import lax
from jax.experimental import pallas as pl
from jax.experimental.pallas import tpu as pltpu
from jax.experimental.pallas import tpu_sc as plsc

N = 10000
E = 320000
D = 128
R = 8

NSC = 2
NT = 16
CH = 128
CW = 16
NG = D // CW
GPC = NG // NSC

CB = 512
BLK = 5120
CR = CB // CH
AHEAD = 4
NBUF = 2 * AHEAD

EPT_COL = E // NT
EPT_CNT = E // (NSC * NT)
NCH_COL = (EPT_COL + CB - 1) // CB
NCH_CNT = (EPT_CNT + CB - 1) // CB
NRND = NCH_COL // AHEAD

SEGS = N * R
DUMP = SEGS
ACC_ROWS = 81920
ZROWS = 256
SEG_PT = SEGS // NT


def _make_sc(with_counts: bool):
    mesh = plsc.VectorSubcoreMesh(core_axis_name="c", subcore_axis_name="s")
    if with_counts:
        out_type = (jax.ShapeDtypeStruct((SEGS, NG, CW), jnp.bfloat16),
                    jax.ShapeDtypeStruct((NSC, SEGS, CW), jnp.bfloat16))
    else:
        out_type = jax.ShapeDtypeStruct((SEGS, NG, CW), jnp.bfloat16)

    def body(xv2, srcg, dstg, etyg, zacc_h, ones_h, *rest):
        if with_counts:
            sums_out, cnt_out = rest[0], rest[1]
            scratch = rest[2:]
        else:
            sums_out = rest[0]
            scratch = rest[1:]
        (stg, gbase, seg3) = scratch[:3]
        rows = scratch[3:3 + NBUF]
        gsem = scratch[3 + NBUF:3 + 2 * NBUF]
        ssem = scratch[3 + 2 * NBUF:3 + 3 * NBUF]

        c = lax.axis_index("c")
        t = lax.axis_index("s")


        def _stage(base_e, nreal, goff):
            nfull = nreal // CB
            ntail = (nreal - nfull * CB) // 16
            CPR = CB // 16
            blocks = []
            done = 0
            while done < nreal:
                take = min(BLK, nreal - done)
                blocks.append((done, take))
                done += take

            def _pass(hbm, write):
                for boff, take in blocks:
                    pltpu.sync_copy(hbm.at[pl.ds(base_e + boff, take)],
                                    stg.at[pl.ds(0, take)])
                    row0 = boff // CB
                    full = take // CB

                    def _r(q, _):
                        for k in range(CPR):
                            write(row0 + q, pl.ds(k * 16, 16),
                                  stg[pl.ds(q * CB + k * 16, 16)])
                        return 0
                    lax.fori_loop(0, full, _r, 0)
                    for k in range((take - full * CB) // 16):
                        write(row0 + full, pl.ds(k * 16, 16),
                              stg[pl.ds(full * CB + k * 16, 16)])

            def _wety(q, sl, v):
                seg3[q, 0, sl] = v * N

            def _wdst(q, sl, v):
                seg3[q, 0, sl] = seg3[q, 0, sl] + v

            _pass(etyg, _wety)
            _pass(dstg, _wdst)
            for k in range(ntail, CB // 16):
                seg3[nfull, 0, pl.ds(k * 16, 16)] = (
                    jnp.full((16,), DUMP, jnp.int32))
            if goff is not None:
                def _wsrc(q, sl, v):
                    gbase[q, 0, sl] = v * NG + goff
                _pass(srcg, _wsrc)
                for k in range(ntail, CB // 16):
                    gbase[nfull, 0, pl.ds(k * 16, 16)] = (
                        jnp.zeros((16,), jnp.int32))

        def _zero_start():
            pltpu.async_copy(
                zacc_h, acc_ref.at[pl.ds(t * (ACC_ROWS // NT), ACC_ROWS // NT)],
                gsem[0])

        def _zero_wait():
            pltpu.make_async_copy(
                zacc_h, acc_ref.at[pl.ds(t * (ACC_ROWS // NT), ACC_ROWS // NT)],
                gsem[0]).wait()

        acc_ref = scratch[3 + 3 * NBUF]

        if with_counts:
            _zero_start()
            pltpu.sync_copy(ones_h, rows[0])
            _stage((c * NT + t) * EPT_CNT, EPT_CNT, None)
            _zero_wait()
            plsc.subcore_barrier()

            def _cfire(ci, _):
                pltpu.async_copy(rows[0], acc_ref.at[seg3.at[ci, 0]], gsem[0],
                                 add=True)
                return 0
            lax.fori_loop(0, NCH_CNT, _cfire, 0)

            def _cdrain(ci, _):
                pltpu.make_async_copy(rows[0], acc_ref.at[seg3.at[0, 0]],
                                      gsem[0]).wait()
                return 0
            lax.fori_loop(0, NCH_CNT, _cdrain, 0)
            plsc.subcore_barrier()
            pltpu.sync_copy(acc_ref.at[pl.ds(t * SEG_PT, SEG_PT)],
                            cnt_out.at[c, pl.ds(t * SEG_PT, SEG_PT)])
            plsc.subcore_barrier()

        _stage(t * EPT_COL, EPT_COL, c * GPC)
        for j in range(GPC):
            p = c * GPC + j
            plsc.subcore_barrier()
            _zero_start()
            if j > 0:
                def _gx(q, _):
                    for k in range(CB // 16):
                        sl = pl.ds(k * 16, 16)
                        gbase[q, 0, sl] = gbase[q, 0, sl] + 1
                    return 0
                lax.fori_loop(0, NCH_COL, _gx, 0)
            _zero_wait()
            plsc.subcore_barrier()

            def _gather(ci, B):
                pltpu.async_copy(xv2.at[gbase.at[ci, 0]], rows[B], gsem[B])

            def _gwait(ci, B):
                pltpu.make_async_copy(xv2.at[gbase.at[ci, 0]], rows[B],
                                      gsem[B]).wait()

            def _scat(ci, B):
                pltpu.async_copy(rows[B], acc_ref.at[seg3.at[ci, 0]], ssem[B],
                                 add=True)

            def _swait(B):
                pltpu.make_async_copy(rows[B], acc_ref.at[seg3.at[0, 0]],
                                      ssem[B]).wait()

            for b in range(AHEAD):
                _gather(b, b)

            def _round(rd, _):
                for b in range(AHEAD):
                    ci = rd * AHEAD + b

                    def _slot(B, B2, ci=ci, rd=rd):
                        _gwait(ci, B)
                        _scat(ci, B)

                        @pl.when(rd < NRND - 1)
                        def _issue_next():
                            @pl.when(rd > 0)
                            def _w():
                                _swait(B2)
                            _gather(ci + AHEAD, B2)

                    @pl.when(rd % 2 == 0)
                    def _even():
                        _slot(b, b + AHEAD)

                    @pl.when(rd % 2 == 1)
                    def _odd():
                        _slot(b + AHEAD, b)
                return 0
            lax.fori_loop(0, NRND, _round, 0)
            for b in range(NBUF):
                _swait(b)
            plsc.subcore_barrier()
            pltpu.sync_copy(acc_ref.at[pl.ds(t * SEG_PT, SEG_PT)],
                            sums_out.at[pl.ds(t * SEG_PT, SEG_PT), p])

    return pl.kernel(
        body,
        out_type=out_type,
        mesh=mesh,
        compiler_params=pltpu.CompilerParams(use_tc_tiling_on_sc=False),
        scratch_types=(
            [
                pltpu.VMEM((BLK,), jnp.int32),
                pltpu.VMEM((NCH_COL, 1, CB), jnp.int32),
                pltpu.VMEM((NCH_COL, 1, CB), jnp.int32),
            ]
            + [pltpu.VMEM((CB, CW), jnp.bfloat16) for _ in range(NBUF)]
            + [pltpu.SemaphoreType.DMA for _ in range(2 * NBUF)]
            + [pltpu.VMEM_SHARED((ACC_ROWS, CW), jnp.bfloat16)]
        ),
    )


_sc_agg_counts = _make_sc(True)
_sc_agg = _make_sc(False)

_BN = 2000


def _rt_body(x_ref, root_ref, b_ref, o_ref):
    o_ref[...] = (jnp.dot(x_ref[...].astype(jnp.float32), root_ref[...],
                          preferred_element_type=jnp.float32) + b_ref[...])


def _make_rt(x_bf16: bool):
    return pl.pallas_call(
        _rt_body,
        grid=(N // _BN,),
        in_specs=[
            pl.BlockSpec((_BN, D), lambda i: (i, 0)),
            pl.BlockSpec((D, D), lambda i: (0, 0)),
            pl.BlockSpec((1, D), lambda i: (0, 0)),
        ],
        out_specs=pl.BlockSpec((_BN, D), lambda i: (i, 0)),
        out_shape=jax.ShapeDtypeStruct((N, D), jnp.float32),
    )


_rt_f32 = _make_rt(False)
_rt_bf16 = _make_rt(True)


def _tc_body(relu, out_bf16, sums_ref, cnt0_ref, cnt1_ref, rt_ref, w_ref,
             o_ref):
    acc = rt_ref[...]
    for r in range(R):
        cnt = (cnt0_ref[r, :, 0:1].astype(jnp.float32)
               + cnt1_ref[r, :, 0:1].astype(jnp.float32))
        inv = 1.0 / jnp.maximum(cnt, 1.0)
        acc = acc + jnp.dot(sums_ref[r].astype(jnp.float32) * inv, w_ref[r],
                            preferred_element_type=jnp.float32)
    if relu:
        acc = jnp.maximum(acc, 0.0)
    if out_bf16:
        acc = acc.astype(jnp.bfloat16)
    o_ref[...] = acc


def _make_tc(relu: bool, out_bf16: bool):
    return pl.pallas_call(
        functools.partial(_tc_body, relu, out_bf16),
        grid=(N // _BN,),
        in_specs=[
            pl.BlockSpec((R, _BN, D), lambda i: (0, i, 0)),
            pl.BlockSpec((R, _BN, CW), lambda i: (0, i, 0)),
            pl.BlockSpec((R, _BN, CW), lambda i: (0, i, 0)),
            pl.BlockSpec((_BN, D), lambda i: (i, 0)),
            pl.BlockSpec((R, D, D), lambda i: (0, 0, 0)),
        ],
        out_specs=pl.BlockSpec((_BN, D), lambda i: (i, 0)),
        out_shape=jax.ShapeDtypeStruct(
            (N, D), jnp.bfloat16 if out_bf16 else jnp.float32),
    )


_tc_layer_relu = _make_tc(True, True)
_tc_layer = _make_tc(False, False)


def kernel(x, edge_index, edge_type, W1, root1, b1, W2, root2, b2):
    src = edge_index[0]
    dst = edge_index[1]

    zacc = jnp.zeros((ACC_ROWS // NT, CW), jnp.bfloat16)
    onesb = jnp.ones((CB, CW), jnp.bfloat16)
    xb = x.astype(jnp.bfloat16)
    sums1, cnt = _sc_agg_counts(xb.reshape(N * NG, CW), src, dst, edge_type,
                                zacc, onesb)
    cnt0 = cnt[0].reshape(R, N, CW)
    cnt1 = cnt[1].reshape(R, N, CW)
    rt1 = _rt_f32(x, root1, b1.reshape(1, D))
    h = _tc_layer_relu(sums1.reshape(R, N, D), cnt0, cnt1, rt1, W1)
    sums2 = _sc_agg(h.reshape(N * NG, CW), src, dst, edge_type, zacc, onesb)
    rt2 = _rt_bf16(h, root2, b2.reshape(1, D))
    out = _tc_layer(sums2.reshape(R, N, D), cnt0, cnt1, rt2, W2)
    return out


# speedup vs baseline: 8.6429x; 1.0014x over previous
import functools

import jax
import jax.numpy as jnp
from jax import lax
from jax.experimental import pallas as pl
from jax.experimental.pallas import tpu as pltpu
from jax.experimental.pallas import tpu_sc as plsc

N = 10000
E = 320000
D = 128
R = 8

NSC = 2
NT = 16
CW = 16
NG = D // CW
GPC = NG // NSC

CB = 512
BLK = 5120
AHEAD = 4
NBUF = 2 * AHEAD

EPT_COL = E // NT
EPT_CNT = E // (NSC * NT)
NCH_COL = (EPT_COL + CB - 1) // CB
NCH_CNT = (EPT_CNT + CB - 1) // CB
NRND = NCH_COL // AHEAD

SEGS = N * R
DUMP = SEGS
ACC_ROWS = 81920
SEG_PT = SEGS // NT


def _make_sc(with_counts: bool):
    mesh = plsc.VectorSubcoreMesh(core_axis_name="c", subcore_axis_name="s")
    if with_counts:
        out_type = (jax.ShapeDtypeStruct((SEGS, NG, CW), jnp.bfloat16),
                    jax.ShapeDtypeStruct((NSC, SEGS, CW), jnp.bfloat16))
    else:
        out_type = jax.ShapeDtypeStruct((SEGS, NG, CW), jnp.bfloat16)

    def body(xv2, srcg, dstg, etyg, zacc_h, ones_h, *rest):
        if with_counts:
            sums_out, cnt_out = rest[0], rest[1]
            scratch = rest[2:]
        else:
            sums_out = rest[0]
            scratch = rest[1:]
        (stg, gbase, seg3) = scratch[:3]
        rows = scratch[3:3 + NBUF]
        gsem = scratch[3 + NBUF:3 + 2 * NBUF]
        ssem = scratch[3 + 2 * NBUF:3 + 3 * NBUF]

        c = lax.axis_index("c")
        t = lax.axis_index("s")


        def _stage(base_e, nreal, goff):
            nfull = nreal // CB
            ntail = (nreal - nfull * CB) // 16
            CPR = CB // 16
            blocks = []
            done = 0
            while done < nreal:
                take = min(BLK, nreal - done)
                blocks.append((done, take))
                done += take

            def _pass(hbm, write):
                for boff, take in blocks:
                    pltpu.sync_copy(hbm.at[pl.ds(base_e + boff, take)],
                                    stg.at[pl.ds(0, take)])
                    row0 = boff // CB
                    full = take // CB

                    def _r(q, _):
                        for k in range(CPR):
                            write(row0 + q, pl.ds(k * 16, 16),
                                  stg[pl.ds(q * CB + k * 16, 16)])
                        return 0
                    lax.fori_loop(0, full, _r, 0)
                    for k in range((take - full * CB) // 16):
                        write(row0 + full, pl.ds(k * 16, 16),
                              stg[pl.ds(full * CB + k * 16, 16)])

            def _wety(q, sl, v):
                seg3[q, 0, sl] = v * N

            def _wdst(q, sl, v):
                seg3[q, 0, sl] = seg3[q, 0, sl] + v

            _pass(etyg, _wety)
            _pass(dstg, _wdst)
            for k in range(ntail, CB // 16):
                seg3[nfull, 0, pl.ds(k * 16, 16)] = (
                    jnp.full((16,), DUMP, jnp.int32))
            if goff is not None:
                def _wsrc(q, sl, v):
                    gbase[q, 0, sl] = v * NG + goff
                _pass(srcg, _wsrc)
                for k in range(ntail, CB // 16):
                    gbase[nfull, 0, pl.ds(k * 16, 16)] = (
                        jnp.zeros((16,), jnp.int32))

        def _zero_start():
            pltpu.async_copy(
                zacc_h, acc_ref.at[pl.ds(t * (ACC_ROWS // NT), ACC_ROWS // NT)],
                gsem[0])

        def _zero_wait():
            pltpu.make_async_copy(
                zacc_h, acc_ref.at[pl.ds(t * (ACC_ROWS // NT), ACC_ROWS // NT)],
                gsem[0]).wait()

        acc_ref = scratch[3 + 3 * NBUF]

        if with_counts:
            _zero_start()
            pltpu.sync_copy(ones_h, rows[0])
            _stage((c * NT + t) * EPT_CNT, EPT_CNT, None)
            _zero_wait()
            plsc.subcore_barrier()

            def _cfire(ci, _):
                pltpu.async_copy(rows[0], acc_ref.at[seg3.at[ci, 0]], gsem[0],
                                 add=True)
                return 0
            lax.fori_loop(0, NCH_CNT, _cfire, 0)

            def _cdrain(ci, _):
                pltpu.make_async_copy(rows[0], acc_ref.at[seg3.at[0, 0]],
                                      gsem[0]).wait()
                return 0
            lax.fori_loop(0, NCH_CNT, _cdrain, 0)
            plsc.subcore_barrier()
            pltpu.sync_copy(acc_ref.at[pl.ds(t * SEG_PT, SEG_PT)],
                            cnt_out.at[c, pl.ds(t * SEG_PT, SEG_PT)])
            plsc.subcore_barrier()

        _stage(t * EPT_COL, EPT_COL, c * GPC)
        for j in range(GPC):
            p = c * GPC + j
            plsc.subcore_barrier()
            _zero_start()
            if j > 0:
                def _gx(q, _):
                    for k in range(CB // 16):
                        sl = pl.ds(k * 16, 16)
                        gbase[q, 0, sl] = gbase[q, 0, sl] + 1
                    return 0
                lax.fori_loop(0, NCH_COL, _gx, 0)
            _zero_wait()
            plsc.subcore_barrier()

            def _gather(ci, B):
                pltpu.async_copy(xv2.at[gbase.at[ci, 0]], rows[B], gsem[B])

            def _gwait(ci, B):
                pltpu.make_async_copy(xv2.at[gbase.at[ci, 0]], rows[B],
                                      gsem[B]).wait()

            def _scat(ci, B):
                pltpu.async_copy(rows[B], acc_ref.at[seg3.at[ci, 0]], ssem[B],
                                 add=True)

            def _swait(B):
                pltpu.make_async_copy(rows[B], acc_ref.at[seg3.at[0, 0]],
                                      ssem[B]).wait()

            for b in range(AHEAD):
                _gather(b, b)

            def _round(rd, _):
                for b in range(AHEAD):
                    ci = rd * AHEAD + b

                    def _slot(B, B2, ci=ci, rd=rd):
                        _gwait(ci, B)
                        _scat(ci, B)

                        @pl.when(rd < NRND - 1)
                        def _issue_next():
                            @pl.when(rd > 0)
                            def _w():
                                _swait(B2)
                            _gather(ci + AHEAD, B2)

                    @pl.when(rd % 2 == 0)
                    def _even():
                        _slot(b, b + AHEAD)

                    @pl.when(rd % 2 == 1)
                    def _odd():
                        _slot(b + AHEAD, b)
                return 0
            lax.fori_loop(0, NRND, _round, 0)
            for b in range(NBUF):
                _swait(b)
            plsc.subcore_barrier()
            pltpu.sync_copy(acc_ref.at[pl.ds(t * SEG_PT, SEG_PT)],
                            sums_out.at[pl.ds(t * SEG_PT, SEG_PT), p])

    return pl.kernel(
        body,
        out_type=out_type,
        mesh=mesh,
        compiler_params=pltpu.CompilerParams(use_tc_tiling_on_sc=False),
        scratch_types=(
            [
                pltpu.VMEM((BLK,), jnp.int32),
                pltpu.VMEM((NCH_COL, 1, CB), jnp.int32),
                pltpu.VMEM((NCH_COL, 1, CB), jnp.int32),
            ]
            + [pltpu.VMEM((CB, CW), jnp.bfloat16) for _ in range(NBUF)]
            + [pltpu.SemaphoreType.DMA for _ in range(2 * NBUF)]
            + [pltpu.VMEM_SHARED((ACC_ROWS, CW), jnp.bfloat16)]
        ),
    )


_sc_agg_counts = _make_sc(True)
_sc_agg = _make_sc(False)

_BN = 2000


def _rt_body(x_ref, root_ref, b_ref, o_ref):
    o_ref[...] = (jnp.dot(x_ref[...].astype(jnp.float32), root_ref[...],
                          preferred_element_type=jnp.float32) + b_ref[...])


def _make_rt(x_bf16: bool):
    return pl.pallas_call(
        _rt_body,
        grid=(N // _BN,),
        in_specs=[
            pl.BlockSpec((_BN, D), lambda i: (i, 0)),
            pl.BlockSpec((D, D), lambda i: (0, 0)),
            pl.BlockSpec((1, D), lambda i: (0, 0)),
        ],
        out_specs=pl.BlockSpec((_BN, D), lambda i: (i, 0)),
        out_shape=jax.ShapeDtypeStruct((N, D), jnp.float32),
    )


_rt_f32 = _make_rt(False)
_rt_bf16 = _make_rt(True)


def _tc_body(relu, out_bf16, sums_ref, cnt0_ref, cnt1_ref, rt_ref, w_ref,
             o_ref):
    acc = rt_ref[...]
    for r in range(R):
        cnt = (cnt0_ref[r, :, 0:1].astype(jnp.float32)
               + cnt1_ref[r, :, 0:1].astype(jnp.float32))
        inv = 1.0 / jnp.maximum(cnt, 1.0)
        acc = acc + jnp.dot(sums_ref[r].astype(jnp.float32) * inv, w_ref[r],
                            preferred_element_type=jnp.float32)
    if relu:
        acc = jnp.maximum(acc, 0.0)
    if out_bf16:
        acc = acc.astype(jnp.bfloat16)
    o_ref[...] = acc


def _make_tc(relu: bool, out_bf16: bool):
    return pl.pallas_call(
        functools.partial(_tc_body, relu, out_bf16),
        grid=(N // _BN,),
        in_specs=[
            pl.BlockSpec((R, _BN, D), lambda i: (0, i, 0)),
            pl.BlockSpec((R, _BN, CW), lambda i: (0, i, 0)),
            pl.BlockSpec((R, _BN, CW), lambda i: (0, i, 0)),
            pl.BlockSpec((_BN, D), lambda i: (i, 0)),
            pl.BlockSpec((R, D, D), lambda i: (0, 0, 0)),
        ],
        out_specs=pl.BlockSpec((_BN, D), lambda i: (i, 0)),
        out_shape=jax.ShapeDtypeStruct(
            (N, D), jnp.bfloat16 if out_bf16 else jnp.float32),
    )


_tc_layer_relu = _make_tc(True, True)
_tc_layer = _make_tc(False, False)


def kernel(x, edge_index, edge_type, W1, root1, b1, W2, root2, b2):
    src = edge_index[0]
    dst = edge_index[1]

    zacc = jnp.zeros((ACC_ROWS // NT, CW), jnp.bfloat16)
    onesb = jnp.ones((CB, CW), jnp.bfloat16)
    xb = x.astype(jnp.bfloat16)
    sums1, cnt = _sc_agg_counts(xb.reshape(N * NG, CW), src, dst, edge_type,
                                zacc, onesb)
    cnt0 = cnt[0].reshape(R, N, CW)
    cnt1 = cnt[1].reshape(R, N, CW)
    rt1 = _rt_f32(x, root1, b1.reshape(1, D))
    h = _tc_layer_relu(sums1.reshape(R, N, D), cnt0, cnt1, rt1, W1)
    sums2 = _sc_agg(h.reshape(N * NG, CW), src, dst, edge_type, zacc, onesb)
    rt2 = _rt_bf16(h, root2, b2.reshape(1, D))
    out = _tc_layer(sums2.reshape(R, N, D), cnt0, cnt1, rt2, W2)
    return out
